# baseline (speedup 1.0000x reference)
import functools

import jax
import jax.numpy as jnp
from jax import lax
from jax.experimental import pallas as pl
from jax.experimental.pallas import tpu as pltpu
from jax.experimental.pallas import tpu_sc as plsc

_NC = 2
_NS = 16
_NW = _NC * _NS


def _sc_add(in2d, pos):
    SB, E = in2d.shape
    S = pos.shape[0]
    B = SB // S
    POS_W = S // _NW
    CH = 8
    NCHUNK = POS_W // CH
    NV = E // 16

    mesh = plsc.VectorSubcoreMesh(core_axis_name="c", subcore_axis_name="s")

    @functools.partial(
        pl.kernel,
        mesh=mesh,
        out_type=jax.ShapeDtypeStruct((SB, E), jnp.float32),
        scratch_types=[
            pltpu.VMEM((CH, E), jnp.float32),
            pltpu.VMEM((CH * B, E), jnp.float32),
        ],
    )
    def body(in_hbm, pos_hbm, out_hbm, pos_v, io_v):
        wid = lax.axis_index("s") * _NC + lax.axis_index("c")
        base_p = wid * POS_W

        def chunk(cidx, carry):
            pbase = base_p + cidx * CH
            rbase = pbase * B
            pltpu.sync_copy(pos_hbm.at[pl.ds(pbase, CH)], pos_v)
            pltpu.sync_copy(in_hbm.at[pl.ds(rbase, CH * B)], io_v)

            def over_p(p, c2):
                def over_v(v, c3):
                    off = v * 16
                    pv = pos_v[p, pl.ds(off, 16)]
                    for b in range(B):
                        r = p * B + b
                        io_v[r, pl.ds(off, 16)] = io_v[r, pl.ds(off, 16)] + pv
                    return c3
                return lax.fori_loop(0, NV, over_v, c2)

            lax.fori_loop(0, CH, over_p, 0)
            pltpu.sync_copy(io_v, out_hbm.at[pl.ds(rbase, CH * B)])
            return carry

        lax.fori_loop(0, NCHUNK, chunk, 0)

    return body(in2d, pos)


def kernel(input, pos_table):
    S, B, E = input.shape
    in2d = input.reshape(S * B, E)
    out2d = _sc_add(in2d, pos_table[:S])
    return out2d.reshape(S, B, E)


# speedup vs baseline: 1.3261x; 1.3261x over previous
import functools

import jax
import jax.numpy as jnp
from jax import lax
from jax.experimental import pallas as pl
from jax.experimental.pallas import tpu as pltpu
from jax.experimental.pallas import tpu_sc as plsc

_NC = 2
_NS = 16
_NW = _NC * _NS


def _sc_add(in2d, pos):
    SB, E = in2d.shape
    S = pos.shape[0]
    B = SB // S
    POS_W = S // _NW
    CH = 8
    NCHUNK = POS_W // CH
    NJ = NCHUNK // 2
    NV = E // 16

    mesh = plsc.VectorSubcoreMesh(core_axis_name="c", subcore_axis_name="s")

    @functools.partial(
        pl.kernel,
        mesh=mesh,
        out_type=jax.ShapeDtypeStruct((SB, E), jnp.float32),
        scratch_types=[
            pltpu.VMEM((CH, E), jnp.float32),
            pltpu.VMEM((CH * B, E), jnp.float32),
            pltpu.VMEM((CH, E), jnp.float32),
            pltpu.VMEM((CH * B, E), jnp.float32),
            pltpu.SemaphoreType.DMA,
            pltpu.SemaphoreType.DMA,
            pltpu.SemaphoreType.DMA,
            pltpu.SemaphoreType.DMA,
            pltpu.SemaphoreType.DMA,
            pltpu.SemaphoreType.DMA,
        ],
    )
    def body(in_hbm, pos_hbm, out_hbm, pos_v0, io_v0, pos_v1, io_v1,
             psem0, isem0, osem0, psem1, isem1, osem1):
        wid = lax.axis_index("s") * _NC + lax.axis_index("c")
        base_c = wid * NCHUNK

        def in_copies(c, pos_b, io_b, psem, isem):
            pbase = c * CH
            rbase = pbase * B
            return (
                pltpu.make_async_copy(pos_hbm.at[pl.ds(pbase, CH)], pos_b, psem),
                pltpu.make_async_copy(in_hbm.at[pl.ds(rbase, CH * B)], io_b, isem),
            )

        def start_in(c, pos_b, io_b, psem, isem):
            ca, cb = in_copies(c, pos_b, io_b, psem, isem)
            ca.start()
            cb.start()

        def wait_in(pos_b, io_b, psem, isem):
            ca, cb = in_copies(base_c, pos_b, io_b, psem, isem)
            ca.wait()
            cb.wait()

        def out_copy(c, io_b, osem):
            rbase = c * CH * B
            return pltpu.make_async_copy(
                io_b, out_hbm.at[pl.ds(rbase, CH * B)], osem)

        def compute(pos_b, io_b):
            @plsc.parallel_loop(0, CH, unroll=2)
            def _(p):
                row = p * B
                for v in range(NV):
                    off = v * 16
                    pv = pos_b[p, pl.ds(off, 16)]
                    for b in range(B):
                        plsc.addupdate(io_b.at[row + b, pl.ds(off, 16)], pv)

        start_in(base_c, pos_v0, io_v0, psem0, isem0)

        def pair(j, carry):
            c0 = base_c + 2 * j
            c1 = c0 + 1
            c2 = c0 + 2

            @pl.when(j > 0)
            def _():
                out_copy(c1, io_v1, osem1).wait()

            start_in(c1, pos_v1, io_v1, psem1, isem1)

            wait_in(pos_v0, io_v0, psem0, isem0)
            compute(pos_v0, io_v0)
            out_copy(c0, io_v0, osem0).start()

            wait_in(pos_v1, io_v1, psem1, isem1)
            compute(pos_v1, io_v1)
            out_copy(c1, io_v1, osem1).start()

            @pl.when(j + 1 < NJ)
            def _():
                out_copy(c0, io_v0, osem0).wait()
                start_in(c2, pos_v0, io_v0, psem0, isem0)

            return carry

        lax.fori_loop(0, NJ, pair, 0)
        out_copy(base_c, io_v0, osem0).wait()
        out_copy(base_c, io_v1, osem1).wait()

    return body(in2d, pos)


def kernel(input, pos_table):
    S, B, E = input.shape
    in2d = input.reshape(S * B, E)
    out2d = _sc_add(in2d, pos_table[:S])
    return out2d.reshape(S, B, E)


# speedup vs baseline: 3.9585x; 2.9851x over previous
import functools

import jax
import jax.numpy as jnp
from jax import lax
from jax.experimental import pallas as pl
from jax.experimental.pallas import tpu as pltpu
from jax.experimental.pallas import tpu_sc as plsc

_NC = 2
_NS = 16
_NW = _NC * _NS


def _sc_add(inp, pos):
    S, B, E = inp.shape
    POS_W = S // _NW
    CH = 8
    NCHUNK = POS_W // CH
    NJ = NCHUNK // 2
    NV = E // 16

    mesh = plsc.VectorSubcoreMesh(core_axis_name="c", subcore_axis_name="s")

    @functools.partial(
        pl.kernel,
        mesh=mesh,
        out_type=jax.ShapeDtypeStruct((S, B, E), jnp.float32),
        scratch_types=[
            pltpu.VMEM((CH, E), jnp.float32),
            pltpu.VMEM((CH, B, E), jnp.float32),
            pltpu.VMEM((CH, E), jnp.float32),
            pltpu.VMEM((CH, B, E), jnp.float32),
            pltpu.SemaphoreType.DMA,
            pltpu.SemaphoreType.DMA,
            pltpu.SemaphoreType.DMA,
            pltpu.SemaphoreType.DMA,
            pltpu.SemaphoreType.DMA,
            pltpu.SemaphoreType.DMA,
        ],
    )
    def body(in_hbm, pos_hbm, out_hbm, pos_v0, io_v0, pos_v1, io_v1,
             psem0, isem0, osem0, psem1, isem1, osem1):
        wid = lax.axis_index("s") * _NC + lax.axis_index("c")
        base_c = wid * NCHUNK

        def in_copies(c, pos_b, io_b, psem, isem):
            pbase = c * CH
            return (
                pltpu.make_async_copy(pos_hbm.at[pl.ds(pbase, CH)], pos_b, psem),
                pltpu.make_async_copy(in_hbm.at[pl.ds(pbase, CH)], io_b, isem),
            )

        def start_in(c, pos_b, io_b, psem, isem):
            ca, cb = in_copies(c, pos_b, io_b, psem, isem)
            ca.start()
            cb.start()

        def wait_in(pos_b, io_b, psem, isem):
            ca, cb = in_copies(base_c, pos_b, io_b, psem, isem)
            ca.wait()
            cb.wait()

        def out_copy(c, io_b, osem):
            pbase = c * CH
            return pltpu.make_async_copy(
                io_b, out_hbm.at[pl.ds(pbase, CH)], osem)

        def compute(pos_b, io_b):
            @plsc.parallel_loop(0, CH, unroll=2)
            def _(p):
                for v in range(NV):
                    off = v * 16
                    pv = pos_b[p, pl.ds(off, 16)]
                    for b in range(B):
                        plsc.addupdate(io_b.at[p, b, pl.ds(off, 16)], pv)

        start_in(base_c, pos_v0, io_v0, psem0, isem0)

        def pair(j, carry):
            c0 = base_c + 2 * j
            c1 = c0 + 1
            c2 = c0 + 2

            @pl.when(j > 0)
            def _():
                out_copy(c1, io_v1, osem1).wait()

            start_in(c1, pos_v1, io_v1, psem1, isem1)

            wait_in(pos_v0, io_v0, psem0, isem0)
            compute(pos_v0, io_v0)
            out_copy(c0, io_v0, osem0).start()

            wait_in(pos_v1, io_v1, psem1, isem1)
            compute(pos_v1, io_v1)
            out_copy(c1, io_v1, osem1).start()

            @pl.when(j + 1 < NJ)
            def _():
                out_copy(c0, io_v0, osem0).wait()
                start_in(c2, pos_v0, io_v0, psem0, isem0)

            return carry

        lax.fori_loop(0, NJ, pair, 0)
        out_copy(base_c, io_v0, osem0).wait()
        out_copy(base_c, io_v1, osem1).wait()

    return body(inp, pos)


def kernel(input, pos_table):
    S, B, E = input.shape
    return _sc_add(input, pos_table[:S])
